# Initial kernel scaffold; baseline (speedup 1.0000x reference)
import jax
import jax.numpy as jnp
from jax.experimental import pallas as pl


def kernel(Xq, edge_indexq, Xt, edge_indext, norm_q, norm_t, u2v_li, node_mask, only_inter, Wq, Wt, Wm):
    raise NotImplementedError("write your pallas kernel here")


# speedup vs baseline: 2.7333x; 2.7333x over previous
import functools

import jax
import jax.numpy as jnp
from jax import lax
from jax.experimental import pallas as pl
from jax.experimental.pallas import tpu as pltpu
from jax.experimental.pallas import tpu_sc as plsc

D = 128
LANES = 16
NC = 2
NS = 16
NW = NC * NS
CH = 128


def _ceil_to(x, m):
    return (x + m - 1) // m * m


def _tc_pre_body(xq, xt, m, wq, wt, wm, y0, y1, y2, y3):
    f32 = jnp.float32
    y0[...] = jnp.dot(xq[...], wq[...], preferred_element_type=f32)
    y1[...] = jnp.dot(xt[...], wt[...], preferred_element_type=f32)
    y2[...] = lax.dot_general(xt[...], wm[...], (((1,), (1,)), ((), ())),
                              preferred_element_type=f32)
    y3[...] = jnp.dot(xq[...] * m[...], wm[...], preferred_element_type=f32)


def _tc_pre(Xq, Xt, maskf, Wq, Wt, Wm, n):
    bn = 1000
    grid = (n // bn,)
    row_spec = pl.BlockSpec((bn, D), lambda i: (i, 0))
    w_spec = pl.BlockSpec((D, D), lambda i: (0, 0))
    m_spec = pl.BlockSpec((bn, 1), lambda i: (i, 0))
    out = jax.ShapeDtypeStruct((n, D), jnp.float32)
    return pl.pallas_call(
        _tc_pre_body,
        grid=grid,
        in_specs=[row_spec, row_spec, m_spec, w_spec, w_spec, w_spec],
        out_specs=[row_spec, row_spec, row_spec, row_spec],
        out_shape=[out, out, out, out],
    )(Xq, Xt, maskf, Wq, Wt, Wm)


def _tc_post_body(p, o):
    o[...] = jnp.maximum(p[:, 0] + p[:, 1], 0.0)


def _tc_post(P, npad):
    bn = 1024
    grid = (2, npad // bn)
    return pl.pallas_call(
        _tc_post_body,
        grid=grid,
        in_specs=[pl.BlockSpec((1, 2, bn, D), lambda g, i: (g, 0, i, 0))],
        out_specs=pl.BlockSpec((1, bn, D), lambda g, i: (g, i, 0)),
        out_shape=jax.ShapeDtypeStruct((2, npad, D), jnp.float32),
    )(P)


def _make_sc(n, npad, ncq, nct, ncx):
    mesh = plsc.VectorSubcoreMesh(
        core_axis_name="c", subcore_axis_name="s", num_cores=NC,
        num_subcores=NS)
    rows_per_sub = npad // NS
    nzchunks = rows_per_sub // CH

    @functools.partial(
        pl.kernel,
        out_type=jax.ShapeDtypeStruct((2, NC, npad, D), jnp.float32),
        mesh=mesh,
        scratch_types=[
            pltpu.VMEM_SHARED((npad, D), jnp.float32),
            pltpu.VMEM((CH,), jnp.int32),
            pltpu.VMEM((CH,), jnp.int32),
            pltpu.VMEM((CH,), jnp.float32),
            pltpu.VMEM((CH, D), jnp.float32),
            pltpu.VMEM((CH, D), jnp.float32),
            pltpu.SemaphoreType.DMA,
        ],
    )
    def sc_kernel(y_qi, y_ti, y_tc, y_qc,
                  eq_src, eq_dst, eq_nrm,
                  et_src, et_dst, et_nrm,
                  xg_q, xs_q, xg_t, xs_t,
                  out,
                  acc, gidx, sidx, nbuf, rows, zbuf, sem):
        c = lax.axis_index("c")
        s = lax.axis_index("s")
        w = s * NC + c

        zv = jnp.zeros((LANES,), jnp.float32)

        def zrow(e, carry):
            for j in range(D // LANES):
                zbuf[e, pl.ds(j * LANES, LANES)] = zv
            return carry

        lax.fori_loop(0, CH, zrow, 0)

        def run_graph(g, nchunks_intra, i_src, i_dst, i_nrm, y_intra,
                      c_g, c_s, y_cross):
            ew_intra = nchunks_intra * CH
            ew_cross = ncx * CH
            for k in range(nzchunks):
                pltpu.sync_copy(
                    zbuf, acc.at[pl.ds(s * rows_per_sub + k * CH, CH)])
            plsc.subcore_barrier()

            def intra_body(i, carry):
                base = w * ew_intra + i * CH
                pltpu.sync_copy(i_src.at[pl.ds(base, CH)], gidx)
                pltpu.sync_copy(i_nrm.at[pl.ds(base, CH)], nbuf)
                pltpu.sync_copy(i_dst.at[pl.ds(base, CH)], sidx)
                pltpu.async_copy(y_intra.at[gidx], rows, sem).wait()

                def scale(e16, cc):
                    nv = nbuf[pl.ds(e16 * LANES, LANES)]
                    for l in range(LANES):
                        nvl = jnp.full((LANES,), nv[l])
                        e = e16 * LANES + l
                        for j in range(D // LANES):
                            sl = pl.ds(j * LANES, LANES)
                            rows[e, sl] = rows[e, sl] * nvl
                    return cc

                lax.fori_loop(0, CH // LANES, scale, 0)
                pltpu.sync_copy(rows, acc.at[sidx], add=True)
                return carry

            lax.fori_loop(0, nchunks_intra, intra_body, 0)

            def cross_body(i, carry):
                base = w * ew_cross + i * CH
                pltpu.sync_copy(c_g.at[pl.ds(base, CH)], gidx)
                pltpu.sync_copy(c_s.at[pl.ds(base, CH)], sidx)
                pltpu.async_copy(y_cross.at[gidx], rows, sem).wait()
                pltpu.sync_copy(rows, acc.at[sidx], add=True)
                return carry

            lax.fori_loop(0, ncx, cross_body, 0)
            plsc.subcore_barrier()

            for k in range(nzchunks):
                off = s * rows_per_sub + k * CH
                pltpu.sync_copy(acc.at[pl.ds(off, CH)], rows)
                pltpu.sync_copy(rows, out.at[g, c, pl.ds(off, CH)])
            plsc.subcore_barrier()

        run_graph(0, ncq, eq_src, eq_dst, eq_nrm, y_qi, xg_q, xs_q, y_tc)
        run_graph(1, nct, et_src, et_dst, et_nrm, y_ti, xg_t, xs_t, y_qc)

    return sc_kernel


def _pad1(a, total, val):
    e = a.shape[0]
    if e == total:
        return a
    return jnp.concatenate([a, jnp.full((total - e,), val, a.dtype)])


def kernel(Xq, edge_indexq, Xt, edge_indext, norm_q, norm_t, u2v_li,
           node_mask, only_inter, Wq, Wt, Wm):
    n = Xq.shape[0]
    npad = _ceil_to(n, NS * CH)
    dump = n + 8

    maskf = node_mask.astype(jnp.float32)[:, None]
    y_qi, y_ti, y_tc, y_qc = _tc_pre(Xq, Xt, maskf, Wq, Wt, Wm, n)

    intra_scale = jnp.where(jnp.asarray(only_inter) != 0, 0.0, 1.0)

    eq = edge_indexq.shape[1]
    et = edge_indext.shape[1]
    ex = u2v_li.shape[1]
    epq = _ceil_to(eq, NW * CH)
    ept = _ceil_to(et, NW * CH)
    epx = _ceil_to(ex, NW * CH)

    eq_src = _pad1(edge_indexq[0], epq, 0)
    eq_dst = _pad1(edge_indexq[1], epq, dump)
    eq_nrm = _pad1(norm_q * intra_scale, epq, 0.0)
    et_src = _pad1(edge_indext[0], ept, 0)
    et_dst = _pad1(edge_indext[1], ept, dump)
    et_nrm = _pad1(norm_t * intra_scale, ept, 0.0)
    u = u2v_li[0]
    v = u2v_li[1]
    xg_q = _pad1(v, epx, 0)
    xs_q = _pad1(u, epx, dump)
    xg_t = _pad1(u, epx, 0)
    xs_t = _pad1(v, epx, dump)

    sc = _make_sc(n, npad, epq // (NW * CH), ept // (NW * CH),
                  epx // (NW * CH))
    P = sc(y_qi, y_ti, y_tc, y_qc,
           eq_src, eq_dst, eq_nrm,
           et_src, et_dst, et_nrm,
           xg_q, xs_q, xg_t, xs_t)

    O = _tc_post(P, npad)
    return (O[0, :n], O[1, :n])


# speedup vs baseline: 3.0832x; 1.1280x over previous
import functools

import jax
import jax.numpy as jnp
from jax import lax
from jax.experimental import pallas as pl
from jax.experimental.pallas import tpu as pltpu
from jax.experimental.pallas import tpu_sc as plsc

D = 128
LANES = 16
NC = 2
NS = 16
CH = 128
SUP = 8
NJ = D // LANES


def _ceil_to(x, m):
    return (x + m - 1) // m * m


def _tc_pre_body(xq, xt, m, wq, wt, wm, y0, y1, y2, y3):
    f32 = jnp.float32
    y0[...] = jnp.dot(xq[...], wq[...], preferred_element_type=f32)
    y1[...] = jnp.dot(xt[...], wt[...], preferred_element_type=f32)
    y2[...] = lax.dot_general(xt[...], wm[...], (((1,), (1,)), ((), ())),
                              preferred_element_type=f32)
    y3[...] = jnp.dot(xq[...] * m[...], wm[...], preferred_element_type=f32)


def _tc_pre(Xq, Xt, maskf, Wq, Wt, Wm, n):
    bn = 1000
    grid = (n // bn,)
    row_spec = pl.BlockSpec((bn, D), lambda i: (i, 0))
    w_spec = pl.BlockSpec((D, D), lambda i: (0, 0))
    m_spec = pl.BlockSpec((bn, 1), lambda i: (i, 0))
    out = jax.ShapeDtypeStruct((n, D), jnp.float32)
    return pl.pallas_call(
        _tc_pre_body,
        grid=grid,
        in_specs=[row_spec, row_spec, m_spec, w_spec, w_spec, w_spec],
        out_specs=[row_spec, row_spec, row_spec, row_spec],
        out_shape=[out, out, out, out],
    )(Xq, Xt, maskf, Wq, Wt, Wm)


def _make_sc(n, npad, nsi, nsx):
    mesh = plsc.VectorSubcoreMesh(
        core_axis_name="c", subcore_axis_name="s", num_cores=NC,
        num_subcores=NS)
    rps = npad // NS
    nz = rps // CH

    @functools.partial(
        pl.kernel,
        out_type=jax.ShapeDtypeStruct((2, npad, D), jnp.float32),
        mesh=mesh,
        scratch_types=[
            pltpu.VMEM_SHARED((npad, D), jnp.float32),
            pltpu.VMEM((SUP, CH), jnp.int32),
            pltpu.VMEM((SUP, CH), jnp.int32),
            pltpu.VMEM((SUP, CH), jnp.float32),
            pltpu.VMEM((CH, D), jnp.float32),
            pltpu.VMEM((CH, D), jnp.float32),
            pltpu.SemaphoreType.DMA,
            pltpu.SemaphoreType.DMA,
        ],
    )
    def sc_kernel(y_qi, y_ti, y_tc, y_qc,
                  eq_src, eq_dst, eq_nrm,
                  et_src, et_dst, et_nrm,
                  xg_q, xs_q, xg_t, xs_t,
                  out,
                  acc, gb, sb, nb, rows0, rows1, sem0, sem1):
        c = lax.axis_index("c")
        s = lax.axis_index("s")
        rows = (rows0, rows1)
        sems = (sem0, sem1)
        zv = jnp.zeros((LANES,), jnp.float32)

        def run_graph(g, y_i, i_src, i_dst, i_nrm, y_c, c_g, c_s):
            def zr(e, cc):
                for j in range(NJ):
                    rows1[e, pl.ds(j * LANES, LANES)] = zv
                return cc

            lax.fori_loop(0, CH, zr, 0)
            for k in range(nz):
                pltpu.sync_copy(rows1, acc.at[pl.ds(s * rps + k * CH, CH)])
            plsc.subcore_barrier()

            def sup_intra(i, cc):
                rb = (s * nsi + i) * SUP
                pltpu.sync_copy(i_src.at[pl.ds(rb, SUP)], gb)
                pltpu.sync_copy(i_dst.at[pl.ds(rb, SUP)], sb)
                pltpu.sync_copy(i_nrm.at[pl.ds(rb, SUP)], nb)
                d = pltpu.async_copy(y_i.at[gb.at[0]], rows[0], sems[0])
                for k in range(SUP):
                    p = k & 1
                    if k + 1 < SUP:
                        dn = pltpu.async_copy(
                            y_i.at[gb.at[k + 1]], rows[1 - p], sems[1 - p])
                    d.wait()
                    buf = rows[p]

                    def scale(e16, c2):
                        nv = nb[k, pl.ds(e16 * LANES, LANES)]
                        for l in range(LANES):
                            nvl = jnp.full((LANES,), nv[l])
                            e = e16 * LANES + l
                            for j in range(NJ):
                                sl = pl.ds(j * LANES, LANES)
                                buf[e, sl] = buf[e, sl] * nvl
                        return c2

                    lax.fori_loop(0, CH // LANES, scale, 0)
                    pltpu.sync_copy(buf, acc.at[sb.at[k]], add=True)
                    if k + 1 < SUP:
                        d = dn
                return cc

            lax.fori_loop(0, nsi, sup_intra, 0)

            def sup_cross(i, cc):
                rb = (s * nsx + i) * SUP
                pltpu.sync_copy(c_g.at[pl.ds(rb, SUP)], gb)
                pltpu.sync_copy(c_s.at[pl.ds(rb, SUP)], sb)
                d = pltpu.async_copy(y_c.at[gb.at[0]], rows[0], sems[0])
                for k in range(SUP):
                    p = k & 1
                    if k + 1 < SUP:
                        dn = pltpu.async_copy(
                            y_c.at[gb.at[k + 1]], rows[1 - p], sems[1 - p])
                    d.wait()
                    pltpu.sync_copy(rows[p], acc.at[sb.at[k]], add=True)
                    if k + 1 < SUP:
                        d = dn
                return cc

            lax.fori_loop(0, nsx, sup_cross, 0)
            plsc.subcore_barrier()

            for k in range(nz):
                off = s * rps + k * CH
                pltpu.sync_copy(acc.at[pl.ds(off, CH)], rows0)

                def rel(e, cc):
                    for j in range(NJ):
                        sl = pl.ds(j * LANES, LANES)
                        rows0[e, sl] = jnp.maximum(rows0[e, sl], 0.0)
                    return cc

                lax.fori_loop(0, CH, rel, 0)
                pltpu.sync_copy(rows0, out.at[g, pl.ds(off, CH)])

        @pl.when(c == 0)
        def _():
            run_graph(0, y_qi, eq_src, eq_dst, eq_nrm, y_tc, xg_q, xs_q)

        @pl.when(c == 1)
        def _():
            run_graph(1, y_ti, et_src, et_dst, et_nrm, y_qc, xg_t, xs_t)

    return sc_kernel


def _pad2d(a, total, val):
    e = a.shape[0]
    if e != total:
        a = jnp.concatenate([a, jnp.full((total - e,), val, a.dtype)])
    return a.reshape(-1, CH)


def kernel(Xq, edge_indexq, Xt, edge_indext, norm_q, norm_t, u2v_li,
           node_mask, only_inter, Wq, Wt, Wm):
    n = Xq.shape[0]
    npad = _ceil_to(n, NS * CH)
    dump = n + 8

    maskf = node_mask.astype(jnp.float32)[:, None]
    y_qi, y_ti, y_tc, y_qc = _tc_pre(Xq, Xt, maskf, Wq, Wt, Wm, n)

    intra_scale = jnp.where(jnp.asarray(only_inter) != 0, 0.0, 1.0)

    unit = NS * SUP * CH
    eq = edge_indexq.shape[1]
    et = edge_indext.shape[1]
    ex = u2v_li.shape[1]
    epq = _ceil_to(eq, unit)
    ept = _ceil_to(et, unit)
    epx = _ceil_to(ex, unit)

    eq_src = _pad2d(edge_indexq[0], epq, 0)
    eq_dst = _pad2d(edge_indexq[1], epq, dump)
    eq_nrm = _pad2d(norm_q * intra_scale, epq, 0.0)
    et_src = _pad2d(edge_indext[0], ept, 0)
    et_dst = _pad2d(edge_indext[1], ept, dump)
    et_nrm = _pad2d(norm_t * intra_scale, ept, 0.0)
    u = u2v_li[0]
    v = u2v_li[1]
    xg_q = _pad2d(v, epx, 0)
    xs_q = _pad2d(u, epx, dump)
    xg_t = _pad2d(u, epx, 0)
    xs_t = _pad2d(v, epx, dump)

    sc = _make_sc(n, npad, epq // unit, epx // unit)
    assert ept == epq
    O = sc(y_qi, y_ti, y_tc, y_qc,
           eq_src, eq_dst, eq_nrm,
           et_src, et_dst, et_nrm,
           xg_q, xs_q, xg_t, xs_t)
    return (O[0, :n], O[1, :n])
